# baseline (speedup 1.0000x reference)
import jax
import jax.numpy as jnp
from jax import lax
from jax.experimental import pallas as pl
from jax.experimental.pallas import tpu as pltpu
from jax.experimental.pallas import tpu_sc as plsc

N = 10000
E = 320000
D = 128
B = 128

NC = 2
NS = 16
DH = D // NC

CH = 256
EPT = 20480
E_PAD = NS * EPT
NCHUNK = EPT // CH
NBUF = 2
N_PAD = 10240
ZPT = N_PAD // NS


def _sc_agg_body(h_hbm, src_hbm, dst_hbm, out_hbm,
                 src_v, dst_v, rows, zbuf, agg_sh, gsems, ssems, zsem):
    cid = lax.axis_index("c")
    sid = lax.axis_index("s")

    with jax.named_scope("zero_stage"):
        ci = pltpu.async_copy(src_hbm.at[sid], src_v, gsems.at[0])
        cj = pltpu.async_copy(dst_hbm.at[sid], dst_v, gsems.at[1])
        def _zrow(i, carry):
            for j in range(DH // 16):
                zbuf[i, pl.ds(j * 16, 16)] = jnp.zeros((16,), jnp.float32)
            return carry
        lax.fori_loop(0, 64, _zrow, 0)
        zbase = sid * ZPT
        for r in range(ZPT // 64):
            pltpu.async_copy(zbuf, agg_sh.at[pl.ds(zbase + r * 64, 64)], zsem)
        ci.wait()
        cj.wait()
        for r in range(ZPT // 64):
            pltpu.make_async_copy(
                zbuf, agg_sh.at[pl.ds(zbase + r * 64, 64)], zsem).wait()
        plsc.subcore_barrier()

    htab = h_hbm.at[cid]

    with jax.named_scope("edges"):
        def _group(q, carry):
            j = NBUF * q
            for b in range(NBUF):
                @pl.when(q > 0)
                def _drain():
                    pltpu.make_async_copy(
                        rows.at[b], agg_sh.at[dst_v.at[j + b]],
                        ssems.at[b]).wait()
                pltpu.async_copy(htab.at[src_v.at[j + b]], rows.at[b],
                                 gsems.at[b])
            for b in range(NBUF):
                pltpu.make_async_copy(
                    htab.at[src_v.at[j + b]], rows.at[b], gsems.at[b]).wait()
                pltpu.async_copy(rows.at[b], agg_sh.at[dst_v.at[j + b]],
                                 ssems.at[b], add=True)
            return carry
        lax.fori_loop(0, NCHUNK // NBUF, _group, 0)
        for b in range(NBUF):
            pltpu.make_async_copy(
                rows.at[b], agg_sh.at[dst_v.at[NCHUNK - NBUF + b]],
                ssems.at[b]).wait()

    plsc.subcore_barrier()
    with jax.named_scope("copyout"):
        obase = sid * ZPT
        pltpu.sync_copy(agg_sh.at[pl.ds(obase, ZPT)],
                        out_hbm.at[cid, pl.ds(obase, ZPT)])


@jax.jit
def _sc_agg(h_split, src_p, dst_p):
    mesh = plsc.VectorSubcoreMesh(core_axis_name="c", subcore_axis_name="s")
    f = pl.kernel(
        _sc_agg_body,
        out_type=jax.ShapeDtypeStruct((NC, N_PAD, DH), jnp.float32),
        mesh=mesh,
        compiler_params=pltpu.CompilerParams(use_tc_tiling_on_sc=False),
        scratch_types=[
            pltpu.VMEM((NCHUNK, CH), jnp.int32),
            pltpu.VMEM((NCHUNK, CH), jnp.int32),
            pltpu.VMEM((NBUF, CH, DH), jnp.float32),
            pltpu.VMEM((64, DH), jnp.float32),
            pltpu.VMEM_SHARED((N_PAD, DH), jnp.float32),
            pltpu.SemaphoreType.DMA((NBUF,)),
            pltpu.SemaphoreType.DMA((NBUF,)),
            pltpu.SemaphoreType.DMA,
        ],
    )
    return f(h_split, src_p, dst_p)


def _dense(p_ref, W_ref, b_ref, g_ref, be_ref):
    s = jnp.concatenate([p_ref[0, :N, :], p_ref[1, :N, :]], axis=1)
    z = jnp.dot(s, W_ref[...], preferred_element_type=jnp.float32) + b_ref[...]
    mu = jnp.mean(z, axis=0, keepdims=True)
    zc = z - mu
    var = jnp.mean(zc * zc, axis=0, keepdims=True)
    y = g_ref[...] * zc * lax.rsqrt(var + 1e-5) + be_ref[...]
    return jnp.maximum(y, 0.0)


def _split_rows(y, out_ref):
    zpad = jnp.zeros((N_PAD - N, DH), jnp.float32)
    for c in range(NC):
        out_ref[c, :N, :] = y[:, c * DH:(c + 1) * DH]
        out_ref[c, N:, :] = zpad


def _tc_dense_body(p_ref, W_ref, b_ref, g_ref, be_ref, out_ref):
    _split_rows(_dense(p_ref, W_ref, b_ref, g_ref, be_ref), out_ref)


def _tc_split_body(x_ref, out_ref):
    _split_rows(x_ref[...], out_ref)


def _tc_dense_pool_body(p_ref, W_ref, b_ref, g_ref, be_ref, batch_ref, out_ref):
    y = _dense(p_ref, W_ref, b_ref, g_ref, be_ref)
    onehot = (batch_ref[...] == lax.broadcasted_iota(jnp.int32, (1, B), 1))
    onehot = onehot.astype(jnp.float32)
    out_ref[...] = lax.dot_general(onehot, y, (((0,), (0,)), ((), ())),
                                   preferred_element_type=jnp.float32)


@jax.jit
def _tc_split(x):
    return pl.pallas_call(
        _tc_split_body,
        out_shape=jax.ShapeDtypeStruct((NC, N_PAD, DH), jnp.float32),
    )(x)


@jax.jit
def _tc_dense(p, W, b, g, be):
    return pl.pallas_call(
        _tc_dense_body,
        out_shape=jax.ShapeDtypeStruct((NC, N_PAD, DH), jnp.float32),
    )(p, W, b, g, be)


@jax.jit
def _tc_dense_pool(p, W, b, g, be, batch2d):
    return pl.pallas_call(
        _tc_dense_pool_body,
        out_shape=jax.ShapeDtypeStruct((B, D), jnp.float32),
    )(p, W, b, g, be, batch2d)


def kernel(x, edge_index, batch, W1, b1, g1, be1, W2, b2, g2, be2):
    src = edge_index[0]
    dst = edge_index[1]
    pad = E_PAD - E
    src_p = jnp.concatenate(
        [src, jnp.zeros((pad,), jnp.int32)]).reshape(NS, NCHUNK, CH)
    dst_p = jnp.concatenate(
        [dst, jnp.full((pad,), N, jnp.int32)]).reshape(NS, NCHUNK, CH)

    b1r, g1r, be1r = b1.reshape(1, D), g1.reshape(1, D), be1.reshape(1, D)
    b2r, g2r, be2r = b2.reshape(1, D), g2.reshape(1, D), be2.reshape(1, D)
    batch2d = batch.reshape(N, 1)

    x_split = _tc_split(x)
    p1 = _sc_agg(x_split, src_p, dst_p)
    h1 = _tc_dense(p1, W1, b1r, g1r, be1r)
    p2 = _sc_agg(h1, src_p, dst_p)
    pooled = _tc_dense_pool(p2, W2, b2r, g2r, be2r, batch2d)
    return (pooled, pooled)


# speedup vs baseline: 1.1401x; 1.1401x over previous
import jax
import jax.numpy as jnp
from jax import lax
from jax.experimental import pallas as pl
from jax.experimental.pallas import tpu as pltpu
from jax.experimental.pallas import tpu_sc as plsc

N = 10000
E = 320000
D = 128
B = 128

NC = 2
NS = 16
DH = D // NC

CH = 128
EPT = 20480
E_PAD = NS * EPT
NCHUNK = EPT // CH
NBUF = 5
N_PAD = 10240
ZPT = N_PAD // NS


def _sc_agg_body(h_hbm, src_hbm, dst_hbm, out_hbm,
                 src_v, dst_v, rows, zbuf, agg_sh, gsems, ssems, zsem):
    cid = lax.axis_index("c")
    sid = lax.axis_index("s")

    with jax.named_scope("zero_stage"):
        ci = pltpu.async_copy(src_hbm.at[sid], src_v, gsems.at[0])
        cj = pltpu.async_copy(dst_hbm.at[sid], dst_v, gsems.at[1])
        def _zrow(i, carry):
            for j in range(DH // 16):
                zbuf[i, pl.ds(j * 16, 16)] = jnp.zeros((16,), jnp.float32)
            return carry
        lax.fori_loop(0, 64, _zrow, 0)
        zbase = sid * ZPT
        for r in range(ZPT // 64):
            pltpu.async_copy(zbuf, agg_sh.at[pl.ds(zbase + r * 64, 64)], zsem)
        ci.wait()
        cj.wait()
        for r in range(ZPT // 64):
            pltpu.make_async_copy(
                zbuf, agg_sh.at[pl.ds(zbase + r * 64, 64)], zsem).wait()
        plsc.subcore_barrier()

    htab = h_hbm.at[cid]

    with jax.named_scope("edges"):
        def _group(q, carry):
            j = NBUF * q
            for b in range(NBUF):
                @pl.when(q > 0)
                def _drain():
                    pltpu.make_async_copy(
                        rows.at[b], agg_sh.at[dst_v.at[j + b]],
                        ssems.at[b]).wait()
                pltpu.async_copy(htab.at[src_v.at[j + b]], rows.at[b],
                                 gsems.at[b])
            for b in range(NBUF):
                pltpu.make_async_copy(
                    htab.at[src_v.at[j + b]], rows.at[b], gsems.at[b]).wait()
                pltpu.async_copy(rows.at[b], agg_sh.at[dst_v.at[j + b]],
                                 ssems.at[b], add=True)
            return carry
        lax.fori_loop(0, NCHUNK // NBUF, _group, 0)
        for b in range(NBUF):
            pltpu.make_async_copy(
                rows.at[b], agg_sh.at[dst_v.at[NCHUNK - NBUF + b]],
                ssems.at[b]).wait()

    plsc.subcore_barrier()
    with jax.named_scope("copyout"):
        obase = sid * ZPT
        pltpu.sync_copy(agg_sh.at[pl.ds(obase, ZPT)],
                        out_hbm.at[cid, pl.ds(obase, ZPT)])


@jax.jit
def _sc_agg(h_split, src_p, dst_p):
    mesh = plsc.VectorSubcoreMesh(core_axis_name="c", subcore_axis_name="s")
    f = pl.kernel(
        _sc_agg_body,
        out_type=jax.ShapeDtypeStruct((NC, N_PAD, DH), jnp.float32),
        mesh=mesh,
        compiler_params=pltpu.CompilerParams(use_tc_tiling_on_sc=False),
        scratch_types=[
            pltpu.VMEM((NCHUNK, CH), jnp.int32),
            pltpu.VMEM((NCHUNK, CH), jnp.int32),
            pltpu.VMEM((NBUF, CH, DH), jnp.float32),
            pltpu.VMEM((64, DH), jnp.float32),
            pltpu.VMEM_SHARED((N_PAD, DH), jnp.float32),
            pltpu.SemaphoreType.DMA((NBUF,)),
            pltpu.SemaphoreType.DMA((NBUF,)),
            pltpu.SemaphoreType.DMA,
        ],
    )
    return f(h_split, src_p, dst_p)


def _dense(p_ref, W_ref, b_ref, g_ref, be_ref):
    s = jnp.concatenate([p_ref[0, :N, :], p_ref[1, :N, :]], axis=1)
    z = jnp.dot(s, W_ref[...], preferred_element_type=jnp.float32) + b_ref[...]
    mu = jnp.mean(z, axis=0, keepdims=True)
    zc = z - mu
    var = jnp.mean(zc * zc, axis=0, keepdims=True)
    y = g_ref[...] * zc * lax.rsqrt(var + 1e-5) + be_ref[...]
    return jnp.maximum(y, 0.0)


def _split_rows(y, out_ref):
    zpad = jnp.zeros((N_PAD - N, DH), jnp.float32)
    for c in range(NC):
        out_ref[c, :N, :] = y[:, c * DH:(c + 1) * DH]
        out_ref[c, N:, :] = zpad


def _tc_dense_body(p_ref, W_ref, b_ref, g_ref, be_ref, out_ref):
    _split_rows(_dense(p_ref, W_ref, b_ref, g_ref, be_ref), out_ref)


def _tc_dense_pool_body(p_ref, W_ref, b_ref, g_ref, be_ref, batch_ref, out_ref):
    y = _dense(p_ref, W_ref, b_ref, g_ref, be_ref)
    onehot = (batch_ref[...] == lax.broadcasted_iota(jnp.int32, (1, B), 1))
    onehot = onehot.astype(jnp.float32)
    out_ref[...] = lax.dot_general(onehot, y, (((0,), (0,)), ((), ())),
                                   preferred_element_type=jnp.float32)


@jax.jit
def _tc_dense(p, W, b, g, be):
    return pl.pallas_call(
        _tc_dense_body,
        out_shape=jax.ShapeDtypeStruct((NC, N_PAD, DH), jnp.float32),
    )(p, W, b, g, be)


@jax.jit
def _tc_dense_pool(p, W, b, g, be, batch2d):
    return pl.pallas_call(
        _tc_dense_pool_body,
        out_shape=jax.ShapeDtypeStruct((B, D), jnp.float32),
    )(p, W, b, g, be, batch2d)


def kernel(x, edge_index, batch, W1, b1, g1, be1, W2, b2, g2, be2):
    src = edge_index[0]
    dst = edge_index[1]
    pad = E_PAD - E
    src_p = jnp.concatenate(
        [src, jnp.zeros((pad,), jnp.int32)]).reshape(NS, NCHUNK, CH)
    dst_p = jnp.concatenate(
        [dst, jnp.full((pad,), N, jnp.int32)]).reshape(NS, NCHUNK, CH)

    b1r, g1r, be1r = b1.reshape(1, D), g1.reshape(1, D), be1.reshape(1, D)
    b2r, g2r, be2r = b2.reshape(1, D), g2.reshape(1, D), be2.reshape(1, D)
    batch2d = batch.reshape(N, 1)

    x_split = jnp.concatenate(
        [x, jnp.zeros((N_PAD - N, D), jnp.float32)]
    ).reshape(N_PAD, NC, DH).transpose(1, 0, 2)
    p1 = _sc_agg(x_split, src_p, dst_p)
    h1 = _tc_dense(p1, W1, b1r, g1r, be1r)
    p2 = _sc_agg(h1, src_p, dst_p)
    pooled = _tc_dense_pool(p2, W2, b2r, g2r, be2r, batch2d)
    return (pooled, pooled)


# speedup vs baseline: 1.1492x; 1.0081x over previous
import jax
import jax.numpy as jnp
from jax import lax
from jax.experimental import pallas as pl
from jax.experimental.pallas import tpu as pltpu
from jax.experimental.pallas import tpu_sc as plsc

N = 10000
E = 320000
D = 128
B = 128

NC = 2
NS = 16
DH = D // NC

CH = 128
EPT = 20480
E_PAD = NS * EPT
NCHUNK = EPT // CH
NBUF = 8
NPH = 2
CPP = NCHUNK // NPH
N_PAD = 10240
ZPT = N_PAD // NS


def _sc_agg_body(h_hbm, src_hbm, dst_hbm, out_hbm,
                 src_v, dst_v, rows, zbuf, agg_sh, gsems, ssems, zsem):
    cid = lax.axis_index("c")
    sid = lax.axis_index("s")

    with jax.named_scope("zero_stage"):
        ci = pltpu.async_copy(src_hbm.at[sid, pl.ds(0, CPP)], src_v,
                              gsems.at[0])
        cj = pltpu.async_copy(dst_hbm.at[sid, pl.ds(0, CPP)], dst_v,
                              gsems.at[1])
        def _zrow(i, carry):
            for j in range(DH // 16):
                zbuf[i, pl.ds(j * 16, 16)] = jnp.zeros((16,), jnp.float32)
            return carry
        lax.fori_loop(0, 32, _zrow, 0)
        zbase = sid * ZPT
        for r in range(ZPT // 32):
            pltpu.async_copy(zbuf, agg_sh.at[pl.ds(zbase + r * 32, 32)], zsem)
        ci.wait()
        cj.wait()
        for r in range(ZPT // 32):
            pltpu.make_async_copy(
                zbuf, agg_sh.at[pl.ds(zbase + r * 32, 32)], zsem).wait()
        plsc.subcore_barrier()

    htab = h_hbm.at[cid]

    with jax.named_scope("edges"):
        for p in range(NPH):
            if p > 0:
                pltpu.sync_copy(src_hbm.at[sid, pl.ds(p * CPP, CPP)], src_v)
                pltpu.sync_copy(dst_hbm.at[sid, pl.ds(p * CPP, CPP)], dst_v)

            def _group(q, carry):
                j = NBUF * q
                for b in range(NBUF):
                    @pl.when(q > 0)
                    def _drain():
                        pltpu.make_async_copy(
                            rows.at[b], agg_sh.at[dst_v.at[j + b]],
                            ssems.at[b]).wait()
                    pltpu.async_copy(htab.at[src_v.at[j + b]], rows.at[b],
                                     gsems.at[b])
                for b in range(NBUF):
                    pltpu.make_async_copy(
                        htab.at[src_v.at[j + b]], rows.at[b],
                        gsems.at[b]).wait()
                    pltpu.async_copy(rows.at[b], agg_sh.at[dst_v.at[j + b]],
                                     ssems.at[b], add=True)
                return carry
            lax.fori_loop(0, CPP // NBUF, _group, 0)
            for b in range(NBUF):
                pltpu.make_async_copy(
                    rows.at[b], agg_sh.at[dst_v.at[CPP - NBUF + b]],
                    ssems.at[b]).wait()

    plsc.subcore_barrier()
    with jax.named_scope("copyout"):
        obase = sid * ZPT
        pltpu.sync_copy(agg_sh.at[pl.ds(obase, ZPT)],
                        out_hbm.at[cid, pl.ds(obase, ZPT)])


@jax.jit
def _sc_agg(h_split, src_p, dst_p):
    mesh = plsc.VectorSubcoreMesh(core_axis_name="c", subcore_axis_name="s")
    f = pl.kernel(
        _sc_agg_body,
        out_type=jax.ShapeDtypeStruct((NC, N_PAD, DH), jnp.float32),
        mesh=mesh,
        compiler_params=pltpu.CompilerParams(use_tc_tiling_on_sc=False),
        scratch_types=[
            pltpu.VMEM((CPP, CH), jnp.int32),
            pltpu.VMEM((CPP, CH), jnp.int32),
            pltpu.VMEM((NBUF, CH, DH), jnp.float32),
            pltpu.VMEM((32, DH), jnp.float32),
            pltpu.VMEM_SHARED((N_PAD, DH), jnp.float32),
            pltpu.SemaphoreType.DMA((NBUF,)),
            pltpu.SemaphoreType.DMA((NBUF,)),
            pltpu.SemaphoreType.DMA,
        ],
    )
    return f(h_split, src_p, dst_p)


def _dense(p_ref, W_ref, b_ref, g_ref, be_ref):
    s = jnp.concatenate([p_ref[0, :N, :], p_ref[1, :N, :]], axis=1)
    z = jnp.dot(s, W_ref[...], preferred_element_type=jnp.float32) + b_ref[...]
    mu = jnp.mean(z, axis=0, keepdims=True)
    zc = z - mu
    var = jnp.mean(zc * zc, axis=0, keepdims=True)
    y = g_ref[...] * zc * lax.rsqrt(var + 1e-5) + be_ref[...]
    return jnp.maximum(y, 0.0)


def _split_rows(y, out_ref):
    zpad = jnp.zeros((N_PAD - N, DH), jnp.float32)
    for c in range(NC):
        out_ref[c, :N, :] = y[:, c * DH:(c + 1) * DH]
        out_ref[c, N:, :] = zpad


def _tc_dense_body(p_ref, W_ref, b_ref, g_ref, be_ref, out_ref):
    _split_rows(_dense(p_ref, W_ref, b_ref, g_ref, be_ref), out_ref)


def _tc_dense_pool_body(p_ref, W_ref, b_ref, g_ref, be_ref, batch_ref, out_ref):
    y = _dense(p_ref, W_ref, b_ref, g_ref, be_ref)
    onehot = (batch_ref[...] == lax.broadcasted_iota(jnp.int32, (1, B), 1))
    onehot = onehot.astype(jnp.float32)
    out_ref[...] = lax.dot_general(onehot, y, (((0,), (0,)), ((), ())),
                                   preferred_element_type=jnp.float32)


@jax.jit
def _tc_dense(p, W, b, g, be):
    return pl.pallas_call(
        _tc_dense_body,
        out_shape=jax.ShapeDtypeStruct((NC, N_PAD, DH), jnp.float32),
    )(p, W, b, g, be)


@jax.jit
def _tc_dense_pool(p, W, b, g, be, batch2d):
    return pl.pallas_call(
        _tc_dense_pool_body,
        out_shape=jax.ShapeDtypeStruct((B, D), jnp.float32),
    )(p, W, b, g, be, batch2d)


def kernel(x, edge_index, batch, W1, b1, g1, be1, W2, b2, g2, be2):
    src = edge_index[0]
    dst = edge_index[1]
    pad = E_PAD - E
    src_p = jnp.concatenate(
        [src, jnp.zeros((pad,), jnp.int32)]).reshape(NS, NCHUNK, CH)
    dst_p = jnp.concatenate(
        [dst, jnp.full((pad,), N, jnp.int32)]).reshape(NS, NCHUNK, CH)

    b1r, g1r, be1r = b1.reshape(1, D), g1.reshape(1, D), be1.reshape(1, D)
    b2r, g2r, be2r = b2.reshape(1, D), g2.reshape(1, D), be2.reshape(1, D)
    batch2d = batch.reshape(N, 1)

    x_split = jnp.concatenate(
        [x, jnp.zeros((N_PAD - N, D), jnp.float32)]
    ).reshape(N_PAD, NC, DH).transpose(1, 0, 2)
    p1 = _sc_agg(x_split, src_p, dst_p)
    h1 = _tc_dense(p1, W1, b1r, g1r, be1r)
    p2 = _sc_agg(h1, src_p, dst_p)
    pooled = _tc_dense_pool(p2, W2, b2r, g2r, be2r, batch2d)
    return (pooled, pooled)
